# baseline (speedup 1.0000x reference)
import jax
import jax.numpy as jnp
from jax.experimental import pallas as pl
from jax.experimental.pallas import tpu as pltpu

E = 8
D = 768
NC = 256
NCHUNKS = D // NC
NH = 2


def _moe_body(onehot_ref, eid_ref, x_ref, W_ref, b_ref, out_ref,
              xm_ref, ws_ref):
    h = pl.program_id(0)
    g = pl.program_id(1)

    @pl.when(jnp.logical_and(g < E, h == 0))
    def _load_w():
        col = pl.multiple_of(g * D, D)
        ws_ref[pl.ds(col, D), :] = W_ref[0].astype(jnp.bfloat16)

    @pl.when(g < E)
    def _build():
        col = pl.multiple_of(g * D, D)
        mask = (eid_ref[...] == g).astype(jnp.float32)
        xm_ref[:, pl.ds(col, D)] = (x_ref[...] * mask).astype(jnp.bfloat16)

    @pl.when(g >= E)
    def _dot():
        ncol = pl.multiple_of((g - E) * NC, NC)
        bias = jnp.dot(onehot_ref[...], b_ref[0],
                       preferred_element_type=jnp.float32)
        out_ref[...] = bias + jnp.dot(xm_ref[...], ws_ref[:, pl.ds(ncol, NC)],
                                      preferred_element_type=jnp.float32)


def kernel(x, curr_video_id, W, b):
    B = x.shape[0]
    BH = B // NH
    eid = curr_video_id.astype(jnp.int32)
    onehot = jax.nn.one_hot(eid, E, dtype=x.dtype)

    return pl.pallas_call(
        _moe_body,
        grid=(NH, E + NCHUNKS),
        in_specs=[
            pl.BlockSpec((BH, E), lambda h, g: (h, 0)),
            pl.BlockSpec((BH, 1), lambda h, g: (h, 0)),
            pl.BlockSpec((BH, D), lambda h, g: (h, 0)),
            pl.BlockSpec((1, D, D),
                         lambda h, g: (jnp.where(h == 0, jnp.minimum(g, E - 1),
                                                 E - 1), 0, 0)),
            pl.BlockSpec((1, E, NC),
                         lambda h, g: (0, 0, jnp.clip(g - E, 0, NCHUNKS - 1))),
        ],
        out_specs=pl.BlockSpec((BH, NC),
                               lambda h, g: (h, jnp.clip(g - E, 0, NCHUNKS - 1))),
        out_shape=jax.ShapeDtypeStruct((B, D), x.dtype),
        scratch_shapes=[
            pltpu.VMEM((BH, E * D), jnp.bfloat16),
            pltpu.VMEM((E * D, D), jnp.bfloat16),
        ],
    )(onehot, eid.reshape(B, 1), x, W, b.reshape(1, E, D))


# speedup vs baseline: 1.2615x; 1.2615x over previous
import jax
import jax.numpy as jnp
from jax.experimental import pallas as pl
from jax.experimental.pallas import tpu as pltpu

E = 8
D = 768
T = 256


def _moe_dense_body(onehot_ref, x_ref, W_ref, b_ref, out_ref):
    oh = onehot_ref[...]
    acc = jnp.dot(oh, b_ref[...], preferred_element_type=jnp.float32)
    x = x_ref[...]
    for e in range(E):
        m = oh[:, e:e + 1]
        acc = acc + jnp.dot(x * m, W_ref[e], preferred_element_type=jnp.float32)
    out_ref[...] = acc


def kernel(x, curr_video_id, W, b):
    B = x.shape[0]
    eid = curr_video_id.astype(jnp.int32)
    onehot = jax.nn.one_hot(eid, E, dtype=x.dtype)
    num_tiles = B // T

    out = pl.pallas_call(
        _moe_dense_body,
        grid=(num_tiles,),
        in_specs=[
            pl.BlockSpec((T, E), lambda t: (t, 0)),
            pl.BlockSpec((T, D), lambda t: (t, 0)),
            pl.BlockSpec((E, D, D), lambda t: (0, 0, 0)),
            pl.BlockSpec((E, D), lambda t: (0, 0)),
        ],
        out_specs=pl.BlockSpec((T, D), lambda t: (t, 0)),
        out_shape=jax.ShapeDtypeStruct((B, D), x.dtype),
    )(onehot, x, W, b)
    return out


# speedup vs baseline: 1.2880x; 1.0210x over previous
import jax
import jax.numpy as jnp
from jax.experimental import pallas as pl
from jax.experimental.pallas import tpu as pltpu

E = 8
D = 768
T = 512


def _moe_dense_body(onehot_ref, x_ref, W_ref, b_ref, out_ref):
    oh = onehot_ref[...]
    acc = jnp.dot(oh, b_ref[...], preferred_element_type=jnp.float32)
    x = x_ref[...]
    for e in range(E):
        m = oh[:, e:e + 1]
        acc = acc + jnp.dot(x * m, W_ref[e], preferred_element_type=jnp.float32)
    out_ref[...] = acc


def kernel(x, curr_video_id, W, b):
    B = x.shape[0]
    eid = curr_video_id.astype(jnp.int32)
    onehot = jax.nn.one_hot(eid, E, dtype=x.dtype)
    num_tiles = B // T

    out = pl.pallas_call(
        _moe_dense_body,
        grid=(num_tiles,),
        in_specs=[
            pl.BlockSpec((T, E), lambda t: (t, 0)),
            pl.BlockSpec((T, D), lambda t: (t, 0)),
            pl.BlockSpec((E, D, D), lambda t: (0, 0, 0)),
            pl.BlockSpec((E, D), lambda t: (0, 0)),
        ],
        out_specs=pl.BlockSpec((T, D), lambda t: (t, 0)),
        out_shape=jax.ShapeDtypeStruct((B, D), x.dtype),
    )(onehot, x, W, b)
    return out
